# baseline (speedup 1.0000x reference)
import functools
import math

import jax
import jax.numpy as jnp
from jax import lax
from jax.experimental import pallas as pl
from jax.experimental.pallas import tpu as pltpu
from jax.experimental.pallas import tpu_sc as plsc

_V = 100000
_N = 1024
_PAD = 0
_SMOOTHING = 0.1
_WB = 2048
_GRID = (_V + _WB - 1) // _WB
_TAIL = _V - (_GRID - 1) * _WB

_NW = 32
_RPW = _N // _NW
_LANES = 16


def _tc_body(tgt8_ref, x1_ref, x2_ref, t_ref, z_ref, n1_ref, acc_ref):
    k = pl.program_id(0)
    x1 = x1_ref[...]
    x2 = x2_ref[...]
    m8 = (tgt8_ref[...] != _PAD).astype(jnp.float32)
    m1 = m8[:, :_N // 2]
    m2 = m8[:, _N // 2:]

    @pl.when(k == 0)
    def _():
        acc_ref[...] = jnp.zeros_like(acc_ref)
        z_ref[0, 0] = (jnp.sum(
            jnp.dot(m1, x1[:, 0:1], preferred_element_type=jnp.float32)) +
            jnp.sum(
            jnp.dot(m2, x2[:, 0:1], preferred_element_type=jnp.float32))) / 8.0
        n1_ref[0, 0] = jnp.sum(m8) / 8.0

    @pl.when(k < _GRID - 1)
    def _():
        acc_ref[...] += (jnp.dot(m1, x1, preferred_element_type=jnp.float32) +
                         jnp.dot(m2, x2, preferred_element_type=jnp.float32))

    @pl.when(k == _GRID - 1)
    def _():
        col = jax.lax.broadcasted_iota(jnp.int32, (_N // 2, _WB), 1)
        xv1 = jnp.where(col < _TAIL, x1, 0.0)
        xv2 = jnp.where(col < _TAIL, x2, 0.0)
        acc_ref[...] += (jnp.dot(m1, xv1, preferred_element_type=jnp.float32) +
                         jnp.dot(m2, xv2, preferred_element_type=jnp.float32))
        t_ref[0, 0] = jnp.sum(acc_ref[...]) / 8.0


def _tc_sums(predicted_target, target8):
    scalar = jax.ShapeDtypeStruct((1, 1), jnp.float32)
    smem = pl.BlockSpec(memory_space=pltpu.MemorySpace.SMEM)
    return pl.pallas_call(
        _tc_body,
        grid=(_GRID,),
        in_specs=[
            pl.BlockSpec((8, _N), lambda k: (0, 0)),
            pl.BlockSpec((_N // 2, _WB), lambda k: (0, k)),
            pl.BlockSpec((_N // 2, _WB), lambda k: (1, k)),
        ],
        out_specs=[smem, smem, smem],
        out_shape=[scalar, scalar, scalar],
        scratch_shapes=[pltpu.VMEM((8, _WB), jnp.float32)],
        compiler_params=pltpu.CompilerParams(
            dimension_semantics=("arbitrary",)),
    )(target8, predicted_target, predicted_target)


_WIN = 128


def _sc_gather_kernel(pt_hbm, tgt_hbm, out_hbm, t_v, c_v, blk_v, acc_v):
    wid = lax.axis_index("s") * 2 + lax.axis_index("c")
    base = wid * _RPW
    pltpu.sync_copy(tgt_hbm.at[pl.ds(base, _RPW)], t_v.at[pl.ds(0, _RPW)])

    for h in range(_RPW // _LANES):
        t = t_v[pl.ds(h * _LANES, _LANES)]
        c_v[pl.ds(h * _LANES, _LANES)] = t & (-128)
        nz = lax.shift_right_logical(t | (0 - t), 31)
        t_v[pl.ds(h * _LANES, _LANES)] = (t & 127) + (nz << 7) - 128

    lane = lax.iota(jnp.int32, _LANES)
    acc_v[...] = jnp.zeros((_LANES,), jnp.float32)

    def body(j, carry):
        o = t_v[pl.ds(j, _LANES)][0]
        c = pl.multiple_of(c_v[pl.ds(j, _LANES)][0], 128)
        r0 = pl.multiple_of(base + (j & (-8)), 8)
        pltpu.sync_copy(pt_hbm.at[pl.ds(r0, 8), pl.ds(c, _WIN)], blk_v)
        rm = j & 7
        off = jnp.full((_LANES,), o, jnp.int32)
        zero = jnp.zeros((_LANES,), jnp.float32)
        val = zero
        for h in range(_WIN // _LANES):
            sel = (lane + h * _LANES) == off
            val = val + jnp.where(sel, blk_v[rm, pl.ds(h * _LANES, _LANES)],
                                  zero)
        acc_v[...] = acc_v[...] + val
        return carry

    lax.fori_loop(0, _RPW, body, 0)
    pltpu.sync_copy(acc_v, out_hbm.at[pl.ds(wid * _LANES, _LANES)])


def _sc_gather(predicted_target, target):
    mesh = plsc.VectorSubcoreMesh(core_axis_name="c", subcore_axis_name="s")
    call = functools.partial(
        pl.kernel,
        mesh=mesh,
        out_type=jax.ShapeDtypeStruct((_NW * _LANES,), jnp.float32),
        scratch_types=[
            pltpu.VMEM((_RPW + _LANES,), jnp.int32),
            pltpu.VMEM((_RPW + _LANES,), jnp.int32),
            pltpu.VMEM((8, _WIN), jnp.float32),
            pltpu.VMEM((_LANES,), jnp.float32),
        ],
    )(_sc_gather_kernel)
    return call(predicted_target, target)


def kernel(predicted_target, target):
    eps = _SMOOTHING / (_V - 2)
    conf = 1.0 - _SMOOTHING
    c_row = (_V - 2) * eps * math.log(eps) + conf * math.log(conf)

    target8 = jnp.broadcast_to(target.reshape(1, _N), (8, _N))
    t_sum, z_sum, n1 = _tc_sums(predicted_target, target8)
    g_parts = _sc_gather(predicted_target, target)

    g_sum = jnp.sum(g_parts)
    loss = (n1[0, 0] * jnp.float32(c_row)
            - jnp.float32(eps) * t_sum[0, 0]
            + jnp.float32(eps) * z_sum[0, 0]
            + jnp.float32(eps - conf) * g_sum)
    return loss


# speedup vs baseline: 1.0031x; 1.0031x over previous
import functools
import math

import jax
import jax.numpy as jnp
from jax import lax
from jax.experimental import pallas as pl
from jax.experimental.pallas import tpu as pltpu
from jax.experimental.pallas import tpu_sc as plsc

_V = 100000
_N = 1024
_PAD = 0
_SMOOTHING = 0.1
_WB = 2048
_GRID = (_V + _WB - 1) // _WB
_TAIL = _V - (_GRID - 1) * _WB

_NW = 32
_RPW = _N // _NW
_LANES = 16


_NBUF = 4
_WLAST = 1664
_VMAIN = 48 * _WB + _WLAST


def _tc_body(tgt8_ref, x_hbm, t_ref, z_ref, n1_ref, buf_ref, sems):
    m8 = (tgt8_ref[...] != _PAD).astype(jnp.float32)

    def copy(k):
        s = k % _NBUF
        w = _WLAST if k == _GRID - 1 else _WB
        return pltpu.make_async_copy(
            x_hbm.at[:, pl.ds(k * _WB, w)],
            buf_ref.at[s, :, pl.ds(0, w)], sems.at[s])

    for k in range(_NBUF):
        copy(k).start()

    acc = jnp.zeros((8, _WB), jnp.float32)
    for k in range(_GRID):
        copy(k).wait()
        x = buf_ref[k % _NBUF]
        if k + _NBUF < _GRID:
            copy(k + _NBUF).start()
        if k == 0:
            z_ref[0, 0] = jnp.sum(
                jnp.dot(m8, x[:, 0:1], preferred_element_type=jnp.float32)) / 8.0
            n1_ref[0, 0] = jnp.sum(m8) / 8.0
        if k == _GRID - 1:
            d = jnp.dot(m8, x[:, :_WLAST], preferred_element_type=jnp.float32)
            acc = acc + jnp.concatenate(
                [d, jnp.zeros((8, _WB - _WLAST), jnp.float32)], axis=1)
        else:
            acc = acc + jnp.dot(m8, x, preferred_element_type=jnp.float32)
    t_ref[0, 0] = jnp.sum(acc) / 8.0


def _tc_sums(predicted_target, target8):
    scalar = jax.ShapeDtypeStruct((1, 1), jnp.float32)
    smem = pl.BlockSpec(memory_space=pltpu.MemorySpace.SMEM)
    return pl.pallas_call(
        _tc_body,
        in_specs=[
            pl.BlockSpec((8, _N), lambda: (0, 0)),
            pl.BlockSpec(memory_space=pltpu.MemorySpace.HBM),
        ],
        out_specs=[smem, smem, smem],
        out_shape=[scalar, scalar, scalar],
        scratch_shapes=[
            pltpu.VMEM((_NBUF, _N, _WB), jnp.float32),
            pltpu.SemaphoreType.DMA((_NBUF,)),
        ],
    )(target8, predicted_target)


_WIN = 128


def _sc_gather_kernel(pt_hbm, tgt_hbm, out_hbm, t_v, c_v, blk_v, acc_v, w_v):
    wid = lax.axis_index("s") * 2 + lax.axis_index("c")
    base = wid * _RPW
    pltpu.sync_copy(tgt_hbm.at[pl.ds(base, _RPW)], t_v.at[pl.ds(0, _RPW)])

    for h in range(_RPW // _LANES):
        t = t_v[pl.ds(h * _LANES, _LANES)]
        c_v[pl.ds(h * _LANES, _LANES)] = t & (-128)
        nz = lax.shift_right_logical(t | (0 - t), 31)
        t_v[pl.ds(h * _LANES, _LANES)] = (t & 127) + (nz << 7) - 128

    lane = lax.iota(jnp.int32, _LANES)
    acc_v[...] = jnp.zeros((_LANES,), jnp.float32)

    w_v[...] = jnp.zeros((_LANES,), jnp.float32)
    for g in range(_RPW // 8):
        rg = pl.multiple_of(base + g * 8, 8)
        vm = pl.multiple_of(0 * base + _VMAIN, 128)
        pltpu.sync_copy(pt_hbm.at[pl.ds(rg, 8), pl.ds(vm, _WIN)], blk_v)
        for r in range(8):
            o = t_v[pl.ds(g * 8 + r, _LANES)][0]
            mr = 1 + (o >> 31)
            mrv = jnp.full((_LANES,), mr, jnp.int32).astype(jnp.float32)
            w_v[...] += (blk_v[r, pl.ds(0, _LANES)] +
                         blk_v[r, pl.ds(_LANES, _LANES)]) * mrv
    pltpu.sync_copy(w_v, out_hbm.at[pl.ds(_NW * _LANES + wid * _LANES, _LANES)])

    def body(j, carry):
        o = t_v[pl.ds(j, _LANES)][0]
        c = pl.multiple_of(c_v[pl.ds(j, _LANES)][0], 128)
        r0 = pl.multiple_of(base + (j & (-8)), 8)
        pltpu.sync_copy(pt_hbm.at[pl.ds(r0, 8), pl.ds(c, _WIN)], blk_v)
        rm = j & 7
        off = jnp.full((_LANES,), o, jnp.int32)
        zero = jnp.zeros((_LANES,), jnp.float32)
        val = zero
        for h in range(_WIN // _LANES):
            sel = (lane + h * _LANES) == off
            val = val + jnp.where(sel, blk_v[rm, pl.ds(h * _LANES, _LANES)],
                                  zero)
        acc_v[...] = acc_v[...] + val
        return carry

    lax.fori_loop(0, _RPW, body, 0)
    pltpu.sync_copy(acc_v, out_hbm.at[pl.ds(wid * _LANES, _LANES)])


def _sc_gather(predicted_target, target):
    mesh = plsc.VectorSubcoreMesh(core_axis_name="c", subcore_axis_name="s")
    call = functools.partial(
        pl.kernel,
        mesh=mesh,
        out_type=jax.ShapeDtypeStruct((2 * _NW * _LANES,), jnp.float32),
        scratch_types=[
            pltpu.VMEM((_RPW + _LANES,), jnp.int32),
            pltpu.VMEM((_RPW + _LANES,), jnp.int32),
            pltpu.VMEM((8, _WIN), jnp.float32),
            pltpu.VMEM((_LANES,), jnp.float32),
            pltpu.VMEM((_LANES,), jnp.float32),
        ],
    )(_sc_gather_kernel)
    return call(predicted_target, target)


def kernel(predicted_target, target):
    eps = _SMOOTHING / (_V - 2)
    conf = 1.0 - _SMOOTHING
    c_row = (_V - 2) * eps * math.log(eps) + conf * math.log(conf)

    target8 = jnp.broadcast_to(target.reshape(1, _N), (8, _N))
    t_sum, z_sum, n1 = _tc_sums(predicted_target, target8)
    g_parts = _sc_gather(predicted_target, target)

    g_sum = jnp.sum(g_parts[:_NW * _LANES])
    w_sum = jnp.sum(g_parts[_NW * _LANES:])
    loss = (n1[0, 0] * jnp.float32(c_row)
            - jnp.float32(eps) * (t_sum[0, 0] + w_sum)
            + jnp.float32(eps) * z_sum[0, 0]
            + jnp.float32(eps - conf) * g_sum)
    return loss
